# Initial kernel scaffold; baseline (speedup 1.0000x reference)
import jax
import jax.numpy as jnp
from jax.experimental import pallas as pl


def kernel(x, edge_index, W1, b1, W2, b2, Wl, bl):
    raise NotImplementedError("write your pallas kernel here")


# speedup vs baseline: 9.4198x; 9.4198x over previous
import jax
import jax.numpy as jnp
from jax import lax
from jax.experimental import pallas as pl
from jax.experimental.pallas import tpu as pltpu
from jax.experimental.pallas import tpu_sc as plsc

N = 100000
E = 1600000
F_IN = 37
H = 64
OUT = 3

LW = 128
CH = 8
ROWS_P = 12544
PAD = ROWS_P * LW - E
RPT = ROWS_P // 16
DROWS_SC = ROWS_P // 2
DRPT = DROWS_SC // 16
NPAD = 100352
STRIPE = NPAD // 16
ZR = 784
R = 1000
GRID = N // R
NSLICE = 4
SLW = H // NSLICE


def _deg_body(dstp_ref, p_ref, deg_s, didx, ones, zbuf, sem):
    del sem
    c = lax.axis_index("c")
    t = lax.axis_index("s")
    zeros16 = jnp.zeros((16,), jnp.float32)
    ones16 = jnp.ones((16,), jnp.float32)
    for i in range(LW // 16):
        ones[pl.ds(16 * i, 16)] = ones16

    def zb(i, _):
        zbuf[pl.ds(i * 16, 16)] = zeros16
        return 0

    lax.fori_loop(0, STRIPE // 16, zb, 0)
    sbase = pl.multiple_of(t * STRIPE, 128)
    pltpu.sync_copy(zbuf, deg_s.at[pl.ds(sbase, STRIPE)])
    plsc.subcore_barrier()

    base = pl.multiple_of(c * DROWS_SC + t * DRPT, 8)

    def ebody(r, _):
        pltpu.sync_copy(dstp_ref.at[pl.ds(pl.multiple_of(base + r * CH, 8), CH)],
                        didx)
        for q in range(CH):
            pltpu.sync_copy(ones, deg_s.at[didx.at[q]], add=True)
        return 0

    lax.fori_loop(0, DRPT // CH, ebody, 0)
    plsc.subcore_barrier()
    obase = pl.multiple_of(c * NPAD + t * STRIPE, 128)
    pltpu.sync_copy(deg_s.at[pl.ds(sbase, STRIPE)],
                    p_ref.at[pl.ds(obase, STRIPE)])


def _agg_body(gf_ref, srcp_ref, dstp_ref, acc_ref, acc_s, sidx, didx, rows,
              zbuf, sem):
    c = lax.axis_index("c")
    t = lax.axis_index("s")
    zeros16 = jnp.zeros((16,), jnp.float32)

    def zb(i, _):
        zbuf[i, :] = zeros16
        return 0

    lax.fori_loop(0, ZR, zb, 0)
    stripe = pl.multiple_of(t * STRIPE, 16)

    for s_loc in range(2):
        sid = 2 * s_loc + c
        for k in range(STRIPE // ZR):
            pltpu.sync_copy(zbuf, acc_s.at[pl.ds(stripe + ZR * k, ZR)])
        plsc.subcore_barrier()

        offv = jnp.full((16,), sid * N, jnp.int32)
        rowbase = pl.multiple_of(t * RPT, 8)

        def ebody(r, _):
            rb = pl.multiple_of(rowbase + r * CH, 8)
            pltpu.sync_copy(srcp_ref.at[pl.ds(rb, CH)], sidx)
            pltpu.sync_copy(dstp_ref.at[pl.ds(rb, CH)], didx)
            for q in range(CH):
                for kk in range(LW // 16):
                    sidx[q, pl.ds(16 * kk, 16)] = (
                        sidx[q, pl.ds(16 * kk, 16)] + offv)
            for q in range(CH):
                pltpu.async_copy(gf_ref.at[sidx.at[q]], rows, sem).wait()
                pltpu.sync_copy(rows, acc_s.at[didx.at[q]], add=True)
            return 0

        lax.fori_loop(0, RPT // CH, ebody, 0)
        plsc.subcore_barrier()

        obase = pl.multiple_of(sid * NPAD + t * STRIPE, 16)
        for k in range(STRIPE // ZR):
            pltpu.sync_copy(acc_s.at[pl.ds(stripe + ZR * k, ZR)],
                            acc_ref.at[pl.ds(obase + ZR * k, ZR)])
        plsc.subcore_barrier()


def _stage_a_body(p_ref, x_ref, w1_ref, g_ref, dinv_ref):
    dinv = lax.rsqrt(p_ref[0] + p_ref[1] + 1.0)
    hw = jnp.dot(x_ref[...], w1_ref[...],
                 preferred_element_type=jnp.float32)
    g = hw * dinv
    for s in range(NSLICE):
        g_ref[s] = g[:, SLW * s:SLW * (s + 1)]
    dinv_ref[...] = dinv


def _stage_b_body(acc_ref, g_ref, dinv_ref, b1_ref, w2_ref, g2_ref):
    dinv = dinv_ref[...]
    hw2 = jnp.zeros((R, H), jnp.float32)
    for s in range(NSLICE):
        hs = jnp.maximum(
            (acc_ref[s] + g_ref[s]) * dinv + b1_ref[0, SLW * s:SLW * (s + 1)],
            0.0)
        hw2 = hw2 + jnp.dot(hs, w2_ref[SLW * s:SLW * (s + 1), :],
                            preferred_element_type=jnp.float32)
    g2 = hw2 * dinv
    for s in range(NSLICE):
        g2_ref[s] = g2[:, SLW * s:SLW * (s + 1)]


def _stage_c_body(acc_ref, g_ref, dinv_ref, b2_ref, wl_ref, bl_ref, o_ref):
    dinv = dinv_ref[...]
    logits = jnp.zeros((R, OUT), jnp.float32) + bl_ref[...]
    for s in range(NSLICE):
        hs = jnp.maximum(
            (acc_ref[s] + g_ref[s]) * dinv + b2_ref[0, SLW * s:SLW * (s + 1)],
            0.0)
        logits = logits + jnp.dot(hs, wl_ref[SLW * s:SLW * (s + 1), :],
                                  preferred_element_type=jnp.float32)
    m = jnp.max(logits, axis=1, keepdims=True)
    e = jnp.exp(logits - m)
    o_ref[...] = e / jnp.sum(e, axis=1, keepdims=True)


_SC_MESH = plsc.VectorSubcoreMesh(core_axis_name="c", subcore_axis_name="s")
_SC_PARAMS = pltpu.CompilerParams(use_tc_tiling_on_sc=False)

_deg_kernel = pl.kernel(
    _deg_body,
    out_type=jax.ShapeDtypeStruct((2 * NPAD,), jnp.float32),
    mesh=_SC_MESH,
    compiler_params=_SC_PARAMS,
    scratch_types=[
        pltpu.MemorySpace.VMEM_SHARED((NPAD,), jnp.float32),
        pltpu.VMEM((CH, LW), jnp.int32),
        pltpu.VMEM((LW,), jnp.float32),
        pltpu.VMEM((STRIPE,), jnp.float32),
        pltpu.SemaphoreType.DMA,
    ],
)

_agg_kernel = pl.kernel(
    _agg_body,
    out_type=jax.ShapeDtypeStruct((NSLICE * NPAD, SLW), jnp.float32),
    mesh=_SC_MESH,
    compiler_params=_SC_PARAMS,
    scratch_types=[
        pltpu.MemorySpace.VMEM_SHARED((NPAD, SLW), jnp.float32),
        pltpu.VMEM((CH, LW), jnp.int32),
        pltpu.VMEM((CH, LW), jnp.int32),
        pltpu.VMEM((LW, SLW), jnp.float32),
        pltpu.VMEM((ZR, SLW), jnp.float32),
        pltpu.SemaphoreType.DMA,
    ],
)

_stage_a = pl.pallas_call(
    _stage_a_body,
    grid=(GRID,),
    in_specs=[
        pl.BlockSpec((2, R, 1), lambda i: (0, i, 0)),
        pl.BlockSpec((R, F_IN), lambda i: (i, 0)),
        pl.BlockSpec((F_IN, H), lambda i: (0, 0)),
    ],
    out_specs=[
        pl.BlockSpec((NSLICE, R, SLW), lambda i: (0, i, 0)),
        pl.BlockSpec((R, 1), lambda i: (i, 0)),
    ],
    out_shape=[
        jax.ShapeDtypeStruct((NSLICE, N, SLW), jnp.float32),
        jax.ShapeDtypeStruct((N, 1), jnp.float32),
    ],
)

_stage_b = pl.pallas_call(
    _stage_b_body,
    grid=(GRID,),
    in_specs=[
        pl.BlockSpec((NSLICE, R, SLW), lambda i: (0, i, 0)),
        pl.BlockSpec((NSLICE, R, SLW), lambda i: (0, i, 0)),
        pl.BlockSpec((R, 1), lambda i: (i, 0)),
        pl.BlockSpec((1, H), lambda i: (0, 0)),
        pl.BlockSpec((H, H), lambda i: (0, 0)),
    ],
    out_specs=pl.BlockSpec((NSLICE, R, SLW), lambda i: (0, i, 0)),
    out_shape=jax.ShapeDtypeStruct((NSLICE, N, SLW), jnp.float32),
)

_stage_c = pl.pallas_call(
    _stage_c_body,
    grid=(GRID,),
    in_specs=[
        pl.BlockSpec((NSLICE, R, SLW), lambda i: (0, i, 0)),
        pl.BlockSpec((NSLICE, R, SLW), lambda i: (0, i, 0)),
        pl.BlockSpec((R, 1), lambda i: (i, 0)),
        pl.BlockSpec((1, H), lambda i: (0, 0)),
        pl.BlockSpec((H, OUT), lambda i: (0, 0)),
        pl.BlockSpec((1, OUT), lambda i: (0, 0)),
    ],
    out_specs=pl.BlockSpec((R, OUT), lambda i: (i, 0)),
    out_shape=jax.ShapeDtypeStruct((N, OUT), jnp.float32),
)


def kernel(x, edge_index, W1, b1, W2, b2, Wl, bl):
    edge_index = edge_index.astype(jnp.int32)
    src = edge_index[0]
    dst = edge_index[1]
    pad_src = jnp.arange(PAD, dtype=jnp.int32) % N
    pad_dst = N + (jnp.arange(PAD, dtype=jnp.int32) % 8)
    srcp = jnp.concatenate([src, pad_src]).reshape(ROWS_P, LW)
    dstp = jnp.concatenate([dst, pad_dst]).reshape(ROWS_P, LW)

    p = _deg_kernel(dstp).reshape(2, NPAD, 1)

    g1, dinv = _stage_a(p, x, W1)
    acc1 = _agg_kernel(g1.reshape(NSLICE * N, SLW), srcp, dstp)
    g2 = _stage_b(acc1.reshape(NSLICE, NPAD, SLW), g1, dinv,
                  b1.reshape(1, H), W2)
    acc2 = _agg_kernel(g2.reshape(NSLICE * N, SLW), srcp, dstp)
    return _stage_c(acc2.reshape(NSLICE, NPAD, SLW), g2, dinv,
                    b2.reshape(1, H), Wl, bl.reshape(1, OUT))


# speedup vs baseline: 15.0680x; 1.5996x over previous
import jax
import jax.numpy as jnp
from jax import lax
from jax.experimental import pallas as pl
from jax.experimental.pallas import tpu as pltpu
from jax.experimental.pallas import tpu_sc as plsc

N = 100000
E = 1600000
F_IN = 37
H = 64
OUT = 3

LW = 128
CE = 512
ROWS_P = 12544
PAD = ROWS_P * LW - E
RPT = ROWS_P // 16
DROWS_SC = ROWS_P // 2
DRPT = DROWS_SC // 16
NPAD = 100352
STRIPE = NPAD // 16
ZR = 392
R = 1000
GRID = N // R
NSLICE = 4
SLW = H // NSLICE


def _deg_body(dstp_ref, p_ref, deg_s, didx, ones, zbuf, sem):
    del sem
    c = lax.axis_index("c")
    t = lax.axis_index("s")
    zeros16 = jnp.zeros((16,), jnp.float32)
    ones16 = jnp.ones((16,), jnp.float32)
    for i in range(CE // 16):
        ones[pl.ds(16 * i, 16)] = ones16

    def zb(i, _):
        zbuf[pl.ds(i * 16, 16)] = zeros16
        return 0

    lax.fori_loop(0, STRIPE // 16, zb, 0)
    sbase = pl.multiple_of(t * STRIPE, 128)
    pltpu.sync_copy(zbuf, deg_s.at[pl.ds(sbase, STRIPE)])
    plsc.subcore_barrier()

    base = pl.multiple_of((c * DROWS_SC + t * DRPT) * LW, 1024)

    def ebody(r, _):
        pltpu.sync_copy(
            dstp_ref.at[pl.ds(pl.multiple_of(base + r * CE, 512), CE)], didx)
        pltpu.sync_copy(ones, deg_s.at[didx], add=True)
        return 0

    lax.fori_loop(0, DRPT * LW // CE, ebody, 0)
    plsc.subcore_barrier()
    obase = pl.multiple_of(c * NPAD + t * STRIPE, 128)
    pltpu.sync_copy(deg_s.at[pl.ds(sbase, STRIPE)],
                    p_ref.at[pl.ds(obase, STRIPE)])


def _agg_body(gf_ref, srcf_ref, dstf_ref, acc_ref, acc_s, sidx, didx, rows,
              zbuf, gsem0, gsem1, ssem0, ssem1):
    c = lax.axis_index("c")
    t = lax.axis_index("s")
    zeros16 = jnp.zeros((16,), jnp.float32)

    def zb(i, _):
        zbuf[i, :] = zeros16
        return 0

    lax.fori_loop(0, ZR, zb, 0)
    stripe = pl.multiple_of(t * STRIPE, 16)

    for s_loc in range(2):
        sid = 2 * s_loc + c
        for k in range(STRIPE // ZR):
            pltpu.sync_copy(zbuf, acc_s.at[pl.ds(stripe + ZR * k, ZR)])
        plsc.subcore_barrier()

        offv = jnp.full((16,), sid * N, jnp.int32)
        ebase = pl.multiple_of(t * RPT * LW, 1024)

        def load_fire(k, buf, gsem):
            off = pl.multiple_of(ebase + k * CE, 512)
            pltpu.sync_copy(srcf_ref.at[pl.ds(off, CE)], sidx.at[buf])
            pltpu.sync_copy(dstf_ref.at[pl.ds(off, CE)], didx.at[buf])
            for kk in range(CE // 16):
                sidx[buf, pl.ds(16 * kk, 16)] = (
                    sidx[buf, pl.ds(16 * kk, 16)] + offv)
            return pltpu.async_copy(gf_ref.at[sidx.at[buf]], rows.at[buf],
                                    gsem)

        def ebody(j, _):
            g0 = load_fire(2 * j, 0, gsem0)
            g1 = load_fire(2 * j + 1, 1, gsem1)
            g0.wait()
            s0 = pltpu.async_copy(rows.at[0], acc_s.at[didx.at[0]], ssem0,
                                  add=True)
            g1.wait()
            s1 = pltpu.async_copy(rows.at[1], acc_s.at[didx.at[1]], ssem1,
                                  add=True)
            s0.wait()
            s1.wait()
            return 0

        lax.fori_loop(0, RPT * LW // (2 * CE), ebody, 0)
        plsc.subcore_barrier()

        obase = pl.multiple_of(sid * NPAD + t * STRIPE, 16)
        for k in range(STRIPE // ZR):
            pltpu.sync_copy(acc_s.at[pl.ds(stripe + ZR * k, ZR)],
                            acc_ref.at[pl.ds(obase + ZR * k, ZR)])
        plsc.subcore_barrier()


def _stage_a_body(p_ref, x_ref, w1_ref, g_ref, dinv_ref):
    dinv = lax.rsqrt(p_ref[0] + p_ref[1] + 1.0)
    hw = jnp.dot(x_ref[...], w1_ref[...],
                 preferred_element_type=jnp.float32)
    g = hw * dinv
    for s in range(NSLICE):
        g_ref[s] = g[:, SLW * s:SLW * (s + 1)]
    dinv_ref[...] = dinv


def _stage_b_body(acc_ref, g_ref, dinv_ref, b1_ref, w2_ref, g2_ref):
    dinv = dinv_ref[...]
    hw2 = jnp.zeros((R, H), jnp.float32)
    for s in range(NSLICE):
        hs = jnp.maximum(
            (acc_ref[s] + g_ref[s]) * dinv + b1_ref[0, SLW * s:SLW * (s + 1)],
            0.0)
        hw2 = hw2 + jnp.dot(hs, w2_ref[SLW * s:SLW * (s + 1), :],
                            preferred_element_type=jnp.float32)
    g2 = hw2 * dinv
    for s in range(NSLICE):
        g2_ref[s] = g2[:, SLW * s:SLW * (s + 1)]


def _stage_c_body(acc_ref, g_ref, dinv_ref, b2_ref, wl_ref, bl_ref, o_ref):
    dinv = dinv_ref[...]
    logits = jnp.zeros((R, OUT), jnp.float32) + bl_ref[...]
    for s in range(NSLICE):
        hs = jnp.maximum(
            (acc_ref[s] + g_ref[s]) * dinv + b2_ref[0, SLW * s:SLW * (s + 1)],
            0.0)
        logits = logits + jnp.dot(hs, wl_ref[SLW * s:SLW * (s + 1), :],
                                  preferred_element_type=jnp.float32)
    m = jnp.max(logits, axis=1, keepdims=True)
    e = jnp.exp(logits - m)
    o_ref[...] = e / jnp.sum(e, axis=1, keepdims=True)


_SC_MESH = plsc.VectorSubcoreMesh(core_axis_name="c", subcore_axis_name="s")
_SC_PARAMS = pltpu.CompilerParams(use_tc_tiling_on_sc=False)

_deg_kernel = pl.kernel(
    _deg_body,
    out_type=jax.ShapeDtypeStruct((2 * NPAD,), jnp.float32),
    mesh=_SC_MESH,
    compiler_params=_SC_PARAMS,
    scratch_types=[
        pltpu.MemorySpace.VMEM_SHARED((NPAD,), jnp.float32),
        pltpu.VMEM((CE,), jnp.int32),
        pltpu.VMEM((CE,), jnp.float32),
        pltpu.VMEM((STRIPE,), jnp.float32),
        pltpu.SemaphoreType.DMA,
    ],
)

_agg_kernel = pl.kernel(
    _agg_body,
    out_type=jax.ShapeDtypeStruct((NSLICE * NPAD, SLW), jnp.float32),
    mesh=_SC_MESH,
    compiler_params=_SC_PARAMS,
    scratch_types=[
        pltpu.MemorySpace.VMEM_SHARED((NPAD, SLW), jnp.float32),
        pltpu.VMEM((2, CE), jnp.int32),
        pltpu.VMEM((2, CE), jnp.int32),
        pltpu.VMEM((2, CE, SLW), jnp.float32),
        pltpu.VMEM((ZR, SLW), jnp.float32),
        pltpu.SemaphoreType.DMA,
        pltpu.SemaphoreType.DMA,
        pltpu.SemaphoreType.DMA,
        pltpu.SemaphoreType.DMA,
    ],
)

_stage_a = pl.pallas_call(
    _stage_a_body,
    grid=(GRID,),
    in_specs=[
        pl.BlockSpec((2, R, 1), lambda i: (0, i, 0)),
        pl.BlockSpec((R, F_IN), lambda i: (i, 0)),
        pl.BlockSpec((F_IN, H), lambda i: (0, 0)),
    ],
    out_specs=[
        pl.BlockSpec((NSLICE, R, SLW), lambda i: (0, i, 0)),
        pl.BlockSpec((R, 1), lambda i: (i, 0)),
    ],
    out_shape=[
        jax.ShapeDtypeStruct((NSLICE, N, SLW), jnp.float32),
        jax.ShapeDtypeStruct((N, 1), jnp.float32),
    ],
)

_stage_b = pl.pallas_call(
    _stage_b_body,
    grid=(GRID,),
    in_specs=[
        pl.BlockSpec((NSLICE, R, SLW), lambda i: (0, i, 0)),
        pl.BlockSpec((NSLICE, R, SLW), lambda i: (0, i, 0)),
        pl.BlockSpec((R, 1), lambda i: (i, 0)),
        pl.BlockSpec((1, H), lambda i: (0, 0)),
        pl.BlockSpec((H, H), lambda i: (0, 0)),
    ],
    out_specs=pl.BlockSpec((NSLICE, R, SLW), lambda i: (0, i, 0)),
    out_shape=jax.ShapeDtypeStruct((NSLICE, N, SLW), jnp.float32),
)

_stage_c = pl.pallas_call(
    _stage_c_body,
    grid=(GRID,),
    in_specs=[
        pl.BlockSpec((NSLICE, R, SLW), lambda i: (0, i, 0)),
        pl.BlockSpec((NSLICE, R, SLW), lambda i: (0, i, 0)),
        pl.BlockSpec((R, 1), lambda i: (i, 0)),
        pl.BlockSpec((1, H), lambda i: (0, 0)),
        pl.BlockSpec((H, OUT), lambda i: (0, 0)),
        pl.BlockSpec((1, OUT), lambda i: (0, 0)),
    ],
    out_specs=pl.BlockSpec((R, OUT), lambda i: (i, 0)),
    out_shape=jax.ShapeDtypeStruct((N, OUT), jnp.float32),
)


def kernel(x, edge_index, W1, b1, W2, b2, Wl, bl):
    edge_index = edge_index.astype(jnp.int32)
    src = edge_index[0]
    dst = edge_index[1]
    pad_src = jnp.arange(PAD, dtype=jnp.int32) % N
    pad_dst = N + (jnp.arange(PAD, dtype=jnp.int32) % 8)
    srcf = jnp.concatenate([src, pad_src])
    dstf = jnp.concatenate([dst, pad_dst])

    p = _deg_kernel(dstf).reshape(2, NPAD, 1)

    g1, dinv = _stage_a(p, x, W1)
    acc1 = _agg_kernel(g1.reshape(NSLICE * N, SLW), srcf, dstf)
    g2 = _stage_b(acc1.reshape(NSLICE, NPAD, SLW), g1, dinv,
                  b1.reshape(1, H), W2)
    acc2 = _agg_kernel(g2.reshape(NSLICE * N, SLW), srcf, dstf)
    return _stage_c(acc2.reshape(NSLICE, NPAD, SLW), g2, dinv,
                    b2.reshape(1, H), Wl, bl.reshape(1, OUT))
